# baseline (speedup 1.0000x reference)
import functools

import jax
import jax.numpy as jnp
from jax import lax
from jax.experimental import pallas as pl
from jax.experimental.pallas import tpu as pltpu
from jax.experimental.pallas import tpu_sc as plsc

_CHANNELS = 64
_N_TOKENS = 16384
_LANES = 16
_MB = 4
_NBUF = 3


def _build_sc_kernel(B, D, C, NC, NS):
    NW = NC * NS
    per_w = B // NW
    n_chunks = per_w // C
    n_mb = C // _MB
    mesh = plsc.VectorSubcoreMesh(core_axis_name="c", subcore_axis_name="s")

    @functools.partial(
        pl.kernel,
        mesh=mesh,
        out_type=jax.ShapeDtypeStruct((3, B // 2, 2 * D), jnp.float32),
        compiler_params=pltpu.CompilerParams(needs_layout_passes=False),
        scratch_types=[
            pltpu.VMEM((C,), jnp.int32),
            pltpu.VMEM((C,), jnp.int32),
            pltpu.VMEM((C,), jnp.int32),
            pltpu.VMEM((C,), jnp.int32),
            pltpu.VMEM((_NBUF, _MB, D, 128), jnp.float32),
            pltpu.VMEM((C, 2 * D), jnp.float32),
            pltpu.VMEM((C, 2 * D), jnp.float32),
            pltpu.VMEM((4, 2 * D), jnp.float32),
            pltpu.VMEM((C // 2, 2 * D), jnp.float32),
            pltpu.VMEM((C // 2, 2 * D), jnp.float32),
            pltpu.VMEM((C // 2, 2 * D), jnp.float32),
            pltpu.SemaphoreType.DMA,
            pltpu.SemaphoreType.DMA,
            pltpu.SemaphoreType.DMA,
            pltpu.SemaphoreType.DMA,
        ],
    )
    def k(xa, xm, xr, xt, waT, wm2, wr2, wt2, out,
          ia, im, ir, it, W, M, T, Rt, As, Ms, Rs, sem, ws0, ws1, ws2):
        wid = lax.axis_index("s") * NC + lax.axis_index("c")
        base0 = wid * per_w
        wsems = [ws0, ws1, ws2]
        rows16 = [jnp.arange(_LANES, dtype=jnp.int32) + j * _LANES
                  for j in range(D // _LANES)]

        pltpu.sync_copy(wr2, Rt)

        def fire_mb(mb, buf):
            off = min(mb * _MB, C - _LANES)
            xv = ia[pl.ds(off, _LANES)]
            for q in range(_MB):
                s = pl.multiple_of((xv[mb * _MB + q - off] >> 7) * 128, 128)
                pltpu.async_copy(
                    waT.at[:, pl.ds(s, 128)], W.at[buf, q], wsems[buf]
                )

        def drain_mb(buf):
            for _ in range(_MB):
                pltpu.make_async_copy(
                    waT.at[:, pl.ds(0, 128)], W.at[buf, 0], wsems[buf]
                ).wait()

        def proc_mb(mb, buf):
            off = min(mb * _MB, C - _LANES)
            xv = ia[pl.ds(off, _LANES)]
            xrs = ir[pl.ds(off, _LANES)]
            for q in range(_MB):
                i = mb * _MB + q
                col = xv[i - off] & 127
                colv = jnp.full((_LANES,), col, dtype=jnp.int32)
                bufv = jnp.full((_LANES,), q, dtype=jnp.int32)
                srow = i // 2
                soff = D * (q & 1)
                rrow = xrs[i - off]
                for j in range(D // _LANES):
                    av = plsc.load_gather(
                        W.at[buf], [bufv, rows16[j], colv])
                    tv = T[i, pl.ds(j * _LANES, _LANES)]
                    mv = M[i, pl.ds(j * _LANES, _LANES)]
                    rv = Rt[rrow, pl.ds(j * _LANES, _LANES)]
                    dsl = pl.ds(soff + j * _LANES, _LANES)
                    As[srow, dsl] = av + tv
                    Ms[srow, dsl] = mv + tv
                    Rs[srow, dsl] = rv + tv

        def chunk(ci, _):
            base = pl.multiple_of(base0 + ci * C, C)
            pltpu.sync_copy(xa.at[pl.ds(base, C)], ia)
            pltpu.sync_copy(xm.at[pl.ds(base, C)], im)
            pltpu.sync_copy(xr.at[pl.ds(base, C)], ir)
            pltpu.sync_copy(xt.at[pl.ds(base, C)], it)
            cps = [
                pltpu.async_copy(wm2.at[im], M, sem),
                pltpu.async_copy(wt2.at[it], T, sem),
            ]
            for b in range(_NBUF):
                fire_mb(b, b)
            for cp in cps:
                cp.wait()
            for mb in range(n_mb):
                buf = mb % _NBUF
                drain_mb(buf)
                proc_mb(mb, buf)
                if mb + _NBUF < n_mb:
                    fire_mb(mb + _NBUF, buf)
            hbase = pl.multiple_of(base // 2, C // 2)
            pltpu.sync_copy(As, out.at[0, pl.ds(hbase, C // 2)])
            pltpu.sync_copy(Ms, out.at[1, pl.ds(hbase, C // 2)])
            pltpu.sync_copy(Rs, out.at[2, pl.ds(hbase, C // 2)])
            return 0

        lax.fori_loop(0, n_chunks, chunk, 0)

    return k


def kernel(x_action, x_mode, x_readout, t, W_action, W_mode, W_readout, W_time):
    info = plsc.get_sparse_core_info()
    k = _build_sc_kernel(_N_TOKENS, _CHANNELS, 64, info.num_cores,
                         info.num_subcores)
    wm2 = jnp.concatenate([W_mode, W_mode], axis=1)
    wr2 = jnp.concatenate([W_readout, W_readout], axis=1)
    wt2 = jnp.concatenate([W_time, W_time], axis=1)
    out128 = k(x_action.astype(jnp.int32), x_mode.astype(jnp.int32),
               x_readout.astype(jnp.int32), t.astype(jnp.int32),
               W_action.T, wm2, wr2, wt2)
    return out128.reshape(3, _N_TOKENS, _CHANNELS)


# speedup vs baseline: 1.0222x; 1.0222x over previous
import functools

import jax
import jax.numpy as jnp
from jax import lax
from jax.experimental import pallas as pl
from jax.experimental.pallas import tpu as pltpu
from jax.experimental.pallas import tpu_sc as plsc

_CHANNELS = 64
_N_TOKENS = 16384
_LANES = 16
_MB = 4
_NBUF = 2


def _build_sc_kernel(B, D, C, NC, NS):
    NW = NC * NS
    per_w = B // NW
    n_chunks = per_w // C
    n_mb = C // _MB
    mesh = plsc.VectorSubcoreMesh(core_axis_name="c", subcore_axis_name="s")

    @functools.partial(
        pl.kernel,
        mesh=mesh,
        out_type=jax.ShapeDtypeStruct((3, B // 2, 2 * D), jnp.float32),
        compiler_params=pltpu.CompilerParams(needs_layout_passes=False),
        scratch_types=[
            pltpu.VMEM((4, C), jnp.int32),
            pltpu.VMEM((_NBUF, _MB, D, 128), jnp.float32),
            pltpu.VMEM((C, 2 * D), jnp.float32),
            pltpu.VMEM((C, 2 * D), jnp.float32),
            pltpu.VMEM((4, 2 * D), jnp.float32),
            pltpu.VMEM((C // 2, 2 * D), jnp.float32),
            pltpu.VMEM((C // 2, 2 * D), jnp.float32),
            pltpu.VMEM((C // 2, 2 * D), jnp.float32),
            pltpu.SemaphoreType.DMA,
            pltpu.SemaphoreType.DMA,
            pltpu.SemaphoreType.DMA,
        ],
    )
    def k(xi, waT, wm2, wr2, wt2, out,
          ix, W, M, T, Rt, As, Ms, Rs, sem, ws0, ws1):
        wid = lax.axis_index("s") * NC + lax.axis_index("c")
        base0 = wid * per_w
        wsems = [ws0, ws1]
        rows16 = [jnp.arange(_LANES, dtype=jnp.int32) + j * _LANES
                  for j in range(D // _LANES)]

        pltpu.sync_copy(wr2, Rt)

        def fire_mb(mb, buf):
            xv = ix[0, pl.ds(mb * _MB, _LANES)]
            for q in range(_MB):
                s = pl.multiple_of((xv[q] >> 7) * 128, 128)
                pltpu.async_copy(
                    waT.at[:, pl.ds(s, 128)], W.at[buf, q], wsems[buf]
                )

        def drain_mb(buf):
            for _ in range(_MB):
                pltpu.make_async_copy(
                    waT.at[:, pl.ds(0, 128)], W.at[buf, 0], wsems[buf]
                ).wait()

        def proc_mb(mb, buf):
            xv = ix[0, pl.ds(mb * _MB, _LANES)]
            xrs = ix[2, pl.ds(mb * _MB, _LANES)]
            for q in range(_MB):
                i = mb * _MB + q
                col = xv[q] & 127
                colv = jnp.full((_LANES,), col, dtype=jnp.int32)
                bufv = jnp.full((_LANES,), q, dtype=jnp.int32)
                srow = i // 2
                soff = D * (q & 1)
                rrow = xrs[q]
                for j in range(D // _LANES):
                    av = plsc.load_gather(
                        W.at[buf], [bufv, rows16[j], colv])
                    tv = T[i, pl.ds(j * _LANES, _LANES)]
                    mv = M[i, pl.ds(j * _LANES, _LANES)]
                    rv = Rt[rrow, pl.ds(j * _LANES, _LANES)]
                    dsl = pl.ds(soff + j * _LANES, _LANES)
                    As[srow, dsl] = av + tv
                    Ms[srow, dsl] = mv + tv
                    Rs[srow, dsl] = rv + tv

        def chunk(ci, _):
            base = pl.multiple_of(base0 + ci * C, C)
            pltpu.sync_copy(xi.at[:, pl.ds(base, C)], ix)
            cps = [
                pltpu.async_copy(wm2.at[ix.at[1]], M, sem),
                pltpu.async_copy(wt2.at[ix.at[3]], T, sem),
            ]
            for b in range(_NBUF):
                fire_mb(b, b)
            for cp in cps:
                cp.wait()

            def body(u, _2):
                mb0 = u * 2
                drain_mb(0)
                proc_mb(mb0, 0)

                @pl.when(mb0 + 2 < n_mb)
                def _f0():
                    fire_mb(mb0 + 2, 0)

                drain_mb(1)
                proc_mb(mb0 + 1, 1)

                @pl.when(mb0 + 3 < n_mb)
                def _f1():
                    fire_mb(mb0 + 3, 1)

                return 0

            lax.fori_loop(0, n_mb // 2, body, 0)
            hbase = pl.multiple_of(base // 2, C // 2)
            pltpu.sync_copy(As, out.at[0, pl.ds(hbase, C // 2)])
            pltpu.sync_copy(Ms, out.at[1, pl.ds(hbase, C // 2)])
            pltpu.sync_copy(Rs, out.at[2, pl.ds(hbase, C // 2)])
            return 0

        lax.fori_loop(0, n_chunks, chunk, 0)

    return k


def kernel(x_action, x_mode, x_readout, t, W_action, W_mode, W_readout, W_time):
    info = plsc.get_sparse_core_info()
    k = _build_sc_kernel(_N_TOKENS, _CHANNELS, 128, info.num_cores,
                         info.num_subcores)
    xi = jnp.stack([x_action.astype(jnp.int32), x_mode.astype(jnp.int32),
                    x_readout.astype(jnp.int32), t.astype(jnp.int32)])
    wm2 = jnp.concatenate([W_mode, W_mode], axis=1)
    wr2 = jnp.concatenate([W_readout, W_readout], axis=1)
    wt2 = jnp.concatenate([W_time, W_time], axis=1)
    out128 = k(xi, W_action.T, wm2, wr2, wt2)
    return out128.reshape(3, _N_TOKENS, _CHANNELS)


# speedup vs baseline: 1.1762x; 1.1506x over previous
import functools

import jax
import jax.numpy as jnp
from jax import lax
from jax.experimental import pallas as pl
from jax.experimental.pallas import tpu as pltpu
from jax.experimental.pallas import tpu_sc as plsc

_CHANNELS = 64
_N_TOKENS = 16384
_LANES = 16
_CAP = 768
_SCAP = 672
_WCAP = 288
_DUMP = _N_TOKENS


def _build_action_kernel(B, D, NC, NS):
    NW = NC * NS
    n_vec = B // _LANES
    mesh = plsc.VectorSubcoreMesh(core_axis_name="c", subcore_axis_name="s")
    out_rows = B + 8 * NW

    @functools.partial(
        pl.kernel,
        mesh=mesh,
        out_type=jax.ShapeDtypeStruct((out_rows, 2 * D), jnp.float32),
        compiler_params=pltpu.CompilerParams(needs_layout_passes=False),
        scratch_types=[
            pltpu.VMEM((B,), jnp.int32),
            pltpu.VMEM((_CAP,), jnp.int32),
            pltpu.VMEM((_CAP,), jnp.int32),
            pltpu.VMEM((_CAP,), jnp.int32),
            pltpu.VMEM((_CAP,), jnp.int32),
            pltpu.VMEM((_WCAP,), jnp.int32),
            pltpu.VMEM((_WCAP,), jnp.int32),
            pltpu.VMEM((2, D, 128), jnp.float32),
            pltpu.VMEM((_SCAP, 2 * D), jnp.float32),
            pltpu.VMEM((7, 96), jnp.int32),
            pltpu.SMEM((256,), jnp.int32),
            pltpu.SemaphoreType.DMA,
            pltpu.SemaphoreType.DMA,
            pltpu.SemaphoreType.DMA,
        ],
    )
    def k1(xa, waT, out, av_all, xs, ps, xs2, ps2, wins, starts, W, St, ps3,
           cnt, sem, ws0, ws1):
        wid = lax.axis_index("s") * NC + lax.axis_index("c")
        wsems = [ws0, ws1]
        iota16 = jnp.arange(_LANES, dtype=jnp.int32)
        dump0 = jnp.int32(_DUMP) + wid * 8
        wbase = (wid * 32768 + 133) // 134
        lane0 = iota16 == 0

        pltpu.sync_copy(xa, av_all)

        def fill(u, _):
            ps2[pl.ds(u * _LANES, _LANES)] = jnp.broadcast_to(
                dump0 + (iota16 & 7), (_LANES,))
            return 0

        lax.fori_loop(0, _CAP // _LANES, fill, 0)

        def zero(d, _):
            cnt[d] = 0
            return 0

        lax.fori_loop(0, 256, zero, 0)
        xs2[pl.ds(0, _LANES)] = jnp.broadcast_to(jnp.int32(-1), (_LANES,))

        def p1(u, off):
            x = av_all[pl.ds(u * _LANES, _LANES)]
            h = ((x >> 7) * 134) >> 15
            m = h == wid
            plsc.store_compressed(xs.at[pl.ds(off, _LANES)], x, mask=m)
            plsc.store_compressed(
                ps.at[pl.ds(off, _LANES)],
                jnp.broadcast_to(jnp.int32(u * _LANES), (_LANES,)) + iota16,
                mask=m)
            return off + plsc.all_reduce_population_count(m)[0]

        n = lax.fori_loop(0, n_vec, p1, jnp.int32(0))

        def hist(i, _):
            wl = (xs[pl.ds(i, _LANES)][0] >> 7) - wbase
            cnt[wl] = cnt[wl] + 1
            return 0

        lax.fori_loop(0, n, hist, 0)

        def pfx(d, run):
            c = cnt[d]
            cnt[d] = run
            return run + c

        lax.fori_loop(0, 256, pfx, jnp.int32(0))

        def sca(i, _):
            x = xs[pl.ds(i, _LANES)][0]
            p = ps[pl.ds(i, _LANES)][0]
            wl = (x >> 7) - wbase
            o = cnt[wl] + _LANES
            cnt[wl] = o + 1 - _LANES
            plsc.store_compressed(
                xs2.at[pl.ds(o, _LANES)],
                jnp.broadcast_to(x, (_LANES,)), mask=lane0)
            plsc.store_compressed(
                ps2.at[pl.ds(o, _LANES)],
                jnp.broadcast_to(p, (_LANES,)), mask=lane0)
            return 0

        lax.fori_loop(0, n, sca, 0)

        def p2(u, off):
            idx = u * _LANES + iota16
            w = xs2[pl.ds(u * _LANES, _LANES)] >> 7
            sel = jnp.maximum(u * _LANES - 1, 0)
            wm1 = xs2[pl.ds(sel, _LANES)] >> 7
            valid = jnp.logical_and(idx >= _LANES, idx < n + _LANES)
            bm = jnp.logical_and(w != wm1, valid)
            plsc.store_compressed(wins.at[pl.ds(off, _LANES)], w, mask=bm)
            plsc.store_compressed(
                starts.at[pl.ds(off, _LANES)],
                jnp.broadcast_to(jnp.int32(u * _LANES), (_LANES,)) + iota16,
                mask=bm)
            return off + plsc.all_reduce_population_count(bm)[0]

        nw = lax.fori_loop(0, _CAP // _LANES, p2, jnp.int32(0))
        starts[pl.ds(nw, _LANES)] = jnp.broadcast_to(n + _LANES, (_LANES,))

        def fire(v, slot):
            wv = wins[pl.ds(v, _LANES)][0]
            s = pl.multiple_of(wv * 128, 128)
            pltpu.async_copy(waT.at[:, pl.ds(s, 128)], W.at[slot],
                             wsems[slot])

        def drain(slot):
            pltpu.make_async_copy(
                waT.at[:, pl.ds(0, 128)], W.at[slot], wsems[slot]).wait()

        def proc(v, slot):
            sv = starts[pl.ds(v, _LANES)]
            s0, s1 = sv[0], sv[1]

            def tok(i, _):
                x = xs2[pl.ds(i, _LANES)][0]
                colv = jnp.broadcast_to(x & 127, (_LANES,))
                for j in range(D // _LANES):
                    av = plsc.load_gather(
                        W.at[slot], [iota16 + j * _LANES, colv])
                    St[i - _LANES, pl.ds(j * _LANES, _LANES)] = av
                return 0

            lax.fori_loop(s0, s1, tok, 0)

        @pl.when(nw > 0)
        def _p0():
            fire(0, 0)

        @pl.when(nw > 1)
        def _p1():
            fire(1, 1)

        def wbody(vv, _):
            v = vv * 2
            drain(0)
            proc(v, 0)

            @pl.when(v + 2 < nw)
            def _f0():
                fire(v + 2, 0)

            @pl.when(v + 1 < nw)
            def _s1():
                drain(1)
                proc(v + 1, 1)

                @pl.when(v + 3 < nw)
                def _f1():
                    fire(v + 3, 1)

            return 0

        lax.fori_loop(0, (nw + 1) // 2, wbody, 0)

        for j in range(7):
            for m in range(6):
                ps3[j, pl.ds(m * _LANES, _LANES)] = (
                    ps2[pl.ds(_LANES + 96 * j + m * _LANES, _LANES)])

        cps = [
            pltpu.async_copy(St.at[pl.ds(96 * j, 96)], out.at[ps3.at[j]],
                             sem)
            for j in range(7)
        ]
        for cp in cps:
            cp.wait()

    return k1


def _build_assemble_kernel(B, D, C, NC, NS, a_rows):
    NW = NC * NS
    per_w = B // NW
    n_chunks = per_w // C
    mesh = plsc.VectorSubcoreMesh(core_axis_name="c", subcore_axis_name="s")

    @functools.partial(
        pl.kernel,
        mesh=mesh,
        out_type=jax.ShapeDtypeStruct((3, B // 2, 2 * D), jnp.float32),
        compiler_params=pltpu.CompilerParams(needs_layout_passes=False),
        scratch_types=[
            pltpu.VMEM((4, C), jnp.int32),
            pltpu.VMEM((C, 2 * D), jnp.float32),
            pltpu.VMEM((C, 2 * D), jnp.float32),
            pltpu.VMEM((C, 2 * D), jnp.float32),
            pltpu.VMEM((4, 2 * D), jnp.float32),
            pltpu.VMEM((C // 2, 2 * D), jnp.float32),
            pltpu.VMEM((C // 2, 2 * D), jnp.float32),
            pltpu.VMEM((C // 2, 2 * D), jnp.float32),
            pltpu.SemaphoreType.DMA,
        ],
    )
    def k2(xi, a_raw, wm2, wr2, wt2, out,
           ix, A, M, T, Rt, As, Ms, Rs, sem):
        wid = lax.axis_index("s") * NC + lax.axis_index("c")
        base0 = wid * per_w
        pltpu.sync_copy(wr2, Rt)

        def chunk(ci, _):
            base = pl.multiple_of(base0 + ci * C, C)
            pltpu.sync_copy(xi.at[:, pl.ds(base, C)], ix)
            cps = [
                pltpu.async_copy(a_raw.at[pl.ds(base, C)], A, sem),
                pltpu.async_copy(wm2.at[ix.at[1]], M, sem),
                pltpu.async_copy(wt2.at[ix.at[3]], T, sem),
            ]
            for cp in cps:
                cp.wait()

            def row(g, _2):
                xrs = ix[2, pl.ds(g * _LANES, _LANES)]
                for l in range(_LANES):
                    i = g * _LANES + l
                    srow = i // 2
                    soff = D * (l & 1)
                    rrow = xrs[l]
                    for j in range(D // _LANES):
                        sl = pl.ds(j * _LANES, _LANES)
                        tv = T[i, sl]
                        av = A[i, sl]
                        mv = M[i, sl]
                        rv = Rt[rrow, sl]
                        dsl = pl.ds(soff + j * _LANES, _LANES)
                        As[srow, dsl] = av + tv
                        Ms[srow, dsl] = mv + tv
                        Rs[srow, dsl] = rv + tv
                return 0

            lax.fori_loop(0, C // _LANES, row, 0)
            hbase = pl.multiple_of(base // 2, C // 2)
            pltpu.sync_copy(As, out.at[0, pl.ds(hbase, C // 2)])
            pltpu.sync_copy(Ms, out.at[1, pl.ds(hbase, C // 2)])
            pltpu.sync_copy(Rs, out.at[2, pl.ds(hbase, C // 2)])
            return 0

        lax.fori_loop(0, n_chunks, chunk, 0)

    return k2


def kernel(x_action, x_mode, x_readout, t, W_action, W_mode, W_readout, W_time):
    info = plsc.get_sparse_core_info()
    NC, NS = info.num_cores, info.num_subcores
    a_rows = _N_TOKENS + 8 * NC * NS
    k1 = _build_action_kernel(_N_TOKENS, _CHANNELS, NC, NS)
    k2 = _build_assemble_kernel(_N_TOKENS, _CHANNELS, 128, NC, NS, a_rows)
    xa = x_action.astype(jnp.int32)
    xi = jnp.stack([xa, x_mode.astype(jnp.int32),
                    x_readout.astype(jnp.int32), t.astype(jnp.int32)])
    wm2 = jnp.concatenate([W_mode, W_mode], axis=1)
    wr2 = jnp.concatenate([W_readout, W_readout], axis=1)
    wt2 = jnp.concatenate([W_time, W_time], axis=1)
    a_raw = k1(xa, W_action.T)
    out128 = k2(xi, a_raw, wm2, wr2, wt2)
    return out128.reshape(3, _N_TOKENS, _CHANNELS)


# speedup vs baseline: 1.4497x; 1.2325x over previous
import functools

import jax
import jax.numpy as jnp
from jax import lax
from jax.experimental import pallas as pl
from jax.experimental.pallas import tpu as pltpu
from jax.experimental.pallas import tpu_sc as plsc

_CHANNELS = 64
_N_TOKENS = 16384
_LANES = 16
_CAP = 768
_SCAP = 672
_WCAP = 288
_DUMP = _N_TOKENS


def _build_action_kernel(B, D, NC, NS):
    NW = NC * NS
    n_vec = B // _LANES
    mesh = plsc.VectorSubcoreMesh(core_axis_name="c", subcore_axis_name="s")
    out_rows = B + 8 * NW

    @functools.partial(
        pl.kernel,
        mesh=mesh,
        out_type=jax.ShapeDtypeStruct((out_rows, 2 * D), jnp.float32),
        compiler_params=pltpu.CompilerParams(needs_layout_passes=False),
        scratch_types=[
            pltpu.VMEM((B // 4,), jnp.int32),
            pltpu.VMEM((_CAP,), jnp.int32),
            pltpu.VMEM((_CAP,), jnp.int32),
            pltpu.VMEM((_CAP,), jnp.int32),
            pltpu.VMEM((_CAP,), jnp.int32),
            pltpu.VMEM((_WCAP,), jnp.int32),
            pltpu.VMEM((_WCAP,), jnp.int32),
            pltpu.VMEM((4, D, 128), jnp.float32),
            pltpu.VMEM((_SCAP, 2 * D), jnp.float32),
            pltpu.VMEM((7, 96), jnp.int32),
            pltpu.SMEM((256,), jnp.int32),
            pltpu.SemaphoreType.DMA,
            pltpu.SemaphoreType.DMA,
            pltpu.SemaphoreType.DMA,
            pltpu.SemaphoreType.DMA,
            pltpu.SemaphoreType.DMA,
        ],
    )
    def k1(xa, waT, out, av_all, xs, ps, xs2, ps2, wins, starts, W, St, ps3,
           cnt, sem, ws0, ws1, ws2, ws3):
        wid = lax.axis_index("s") * NC + lax.axis_index("c")
        wsems = [ws0, ws1, ws2, ws3]
        iota16 = jnp.arange(_LANES, dtype=jnp.int32)
        dump0 = jnp.int32(_DUMP) + wid * 8
        wbase = (wid * 32768 + 133) // 134
        lane0 = iota16 == 0

        def fill(u, _):
            ps2[pl.ds(u * _LANES, _LANES)] = jnp.broadcast_to(
                dump0 + (iota16 & 7), (_LANES,))
            return 0

        lax.fori_loop(0, _CAP // _LANES, fill, 0)

        def zero(d, _):
            cnt[d] = 0
            return 0

        lax.fori_loop(0, 256, zero, 0)
        xs2[pl.ds(0, _LANES)] = jnp.broadcast_to(jnp.int32(-1), (_LANES,))

        def p1c(c, off_c):
            cb = pl.multiple_of(c * (B // 4), B // 4)
            pltpu.sync_copy(xa.at[pl.ds(cb, B // 4)], av_all)

            def p1(u, off):
                x = av_all[pl.ds(u * _LANES, _LANES)]
                h = ((x >> 7) * 134) >> 15
                m = h == wid
                plsc.store_compressed(xs.at[pl.ds(off, _LANES)], x, mask=m)
                plsc.store_compressed(
                    ps.at[pl.ds(off, _LANES)],
                    jnp.broadcast_to(c * (B // 4) + u * _LANES, (_LANES,))
                    + iota16, mask=m)
                return off + plsc.all_reduce_population_count(m)[0]

            return lax.fori_loop(0, n_vec // 4, p1, off_c)

        n = lax.fori_loop(0, 4, p1c, jnp.int32(0))

        def hist(i, _):
            wl = (xs[pl.ds(i, _LANES)][0] >> 7) - wbase
            cnt[wl] = cnt[wl] + 1
            return 0

        lax.fori_loop(0, n, hist, 0)

        def pfx(d, run):
            c = cnt[d]
            cnt[d] = run
            return run + c

        lax.fori_loop(0, 256, pfx, jnp.int32(0))

        def sca(i, _):
            x = xs[pl.ds(i, _LANES)][0]
            p = ps[pl.ds(i, _LANES)][0]
            wl = (x >> 7) - wbase
            o = cnt[wl] + _LANES
            cnt[wl] = o + 1 - _LANES
            plsc.store_compressed(
                xs2.at[pl.ds(o, _LANES)],
                jnp.broadcast_to(x, (_LANES,)), mask=lane0)
            plsc.store_compressed(
                ps2.at[pl.ds(o, _LANES)],
                jnp.broadcast_to(p, (_LANES,)), mask=lane0)
            return 0

        lax.fori_loop(0, n, sca, 0)

        def p2(u, off):
            idx = u * _LANES + iota16
            w = xs2[pl.ds(u * _LANES, _LANES)] >> 7
            sel = jnp.maximum(u * _LANES - 1, 0)
            wm1 = xs2[pl.ds(sel, _LANES)] >> 7
            valid = jnp.logical_and(idx >= _LANES, idx < n + _LANES)
            bm = jnp.logical_and(w != wm1, valid)
            plsc.store_compressed(wins.at[pl.ds(off, _LANES)], w, mask=bm)
            plsc.store_compressed(
                starts.at[pl.ds(off, _LANES)],
                jnp.broadcast_to(jnp.int32(u * _LANES), (_LANES,)) + iota16,
                mask=bm)
            return off + plsc.all_reduce_population_count(bm)[0]

        nw = lax.fori_loop(0, _CAP // _LANES, p2, jnp.int32(0))
        starts[pl.ds(nw, _LANES)] = jnp.broadcast_to(n + _LANES, (_LANES,))

        def fire(v, slot):
            wv = wins[pl.ds(v, _LANES)][0]
            s = pl.multiple_of(wv * 128, 128)
            pltpu.async_copy(waT.at[:, pl.ds(s, 128)], W.at[slot],
                             wsems[slot])

        def drain(slot):
            pltpu.make_async_copy(
                waT.at[:, pl.ds(0, 128)], W.at[slot], wsems[slot]).wait()

        def proc(v, slot):
            sv = starts[pl.ds(v, _LANES)]
            s0, s1 = sv[0], sv[1]

            def tok(i, _):
                x = xs2[pl.ds(i, _LANES)][0]
                colv = jnp.broadcast_to(x & 127, (_LANES,))
                for j in range(D // _LANES):
                    av = plsc.load_gather(
                        W.at[slot], [iota16 + j * _LANES, colv])
                    St[i - _LANES, pl.ds(j * _LANES, _LANES)] = av
                return 0

            lax.fori_loop(s0, s1, tok, 0)

        for b in range(4):
            @pl.when(nw > b)
            def _pro(b=b):
                fire(b, b)

        def wbody(vv, _):
            v0 = vv * 4
            drain(0)
            proc(v0, 0)

            @pl.when(v0 + 4 < nw)
            def _f0():
                fire(v0 + 4, 0)

            for b in range(1, 4):
                @pl.when(v0 + b < nw)
                def _sb(b=b):
                    drain(b)
                    proc(v0 + b, b)

                    @pl.when(v0 + b + 4 < nw)
                    def _fb(b=b):
                        fire(v0 + b + 4, b)

            return 0

        lax.fori_loop(0, (nw + 3) // 4, wbody, 0)

        for j in range(7):
            for m in range(6):
                ps3[j, pl.ds(m * _LANES, _LANES)] = (
                    ps2[pl.ds(_LANES + 96 * j + m * _LANES, _LANES)])

        cps = [
            pltpu.async_copy(St.at[pl.ds(96 * j, 96)], out.at[ps3.at[j]],
                             sem)
            for j in range(7)
        ]
        for cp in cps:
            cp.wait()

    return k1


def _build_assemble_kernel(B, D, C, NC, NS, a_rows):
    NW = NC * NS
    per_w = B // NW
    n_chunks = per_w // C
    mesh = plsc.VectorSubcoreMesh(core_axis_name="c", subcore_axis_name="s")

    @functools.partial(
        pl.kernel,
        mesh=mesh,
        out_type=jax.ShapeDtypeStruct((3, B // 2, 2 * D), jnp.float32),
        compiler_params=pltpu.CompilerParams(needs_layout_passes=False),
        scratch_types=[
            pltpu.VMEM((4, C), jnp.int32),
            pltpu.VMEM((C, 2 * D), jnp.float32),
            pltpu.VMEM((C, 2 * D), jnp.float32),
            pltpu.VMEM((C, 2 * D), jnp.float32),
            pltpu.VMEM((4, 2 * D), jnp.float32),
            pltpu.VMEM((C // 2, 2 * D), jnp.float32),
            pltpu.VMEM((C // 2, 2 * D), jnp.float32),
            pltpu.VMEM((C // 2, 2 * D), jnp.float32),
            pltpu.SemaphoreType.DMA,
        ],
    )
    def k2(xi, a_raw, wm2, wr2, wt2, out,
           ix, A, M, T, Rt, As, Ms, Rs, sem):
        wid = lax.axis_index("s") * NC + lax.axis_index("c")
        base0 = wid * per_w
        pltpu.sync_copy(wr2, Rt)

        def chunk(ci, _):
            base = pl.multiple_of(base0 + ci * C, C)
            pltpu.sync_copy(xi.at[:, pl.ds(base, C)], ix)
            cps = [
                pltpu.async_copy(a_raw.at[pl.ds(base, C)], A, sem),
                pltpu.async_copy(wm2.at[ix.at[1]], M, sem),
                pltpu.async_copy(wt2.at[ix.at[3]], T, sem),
            ]
            for cp in cps:
                cp.wait()

            def row(g, _2):
                xrs = ix[2, pl.ds(g * _LANES, _LANES)]
                for l in range(_LANES):
                    i = g * _LANES + l
                    srow = i // 2
                    soff = D * (l & 1)
                    rrow = xrs[l]
                    for j in range(D // _LANES):
                        sl = pl.ds(j * _LANES, _LANES)
                        tv = T[i, sl]
                        av = A[i, sl]
                        mv = M[i, sl]
                        rv = Rt[rrow, sl]
                        dsl = pl.ds(soff + j * _LANES, _LANES)
                        As[srow, dsl] = av + tv
                        Ms[srow, dsl] = mv + tv
                        Rs[srow, dsl] = rv + tv
                return 0

            lax.fori_loop(0, C // _LANES, row, 0)
            hbase = pl.multiple_of(base // 2, C // 2)
            pltpu.sync_copy(As, out.at[0, pl.ds(hbase, C // 2)])
            pltpu.sync_copy(Ms, out.at[1, pl.ds(hbase, C // 2)])
            pltpu.sync_copy(Rs, out.at[2, pl.ds(hbase, C // 2)])
            return 0

        lax.fori_loop(0, n_chunks, chunk, 0)

    return k2


def kernel(x_action, x_mode, x_readout, t, W_action, W_mode, W_readout, W_time):
    info = plsc.get_sparse_core_info()
    NC, NS = info.num_cores, info.num_subcores
    a_rows = _N_TOKENS + 8 * NC * NS
    k1 = _build_action_kernel(_N_TOKENS, _CHANNELS, NC, NS)
    k2 = _build_assemble_kernel(_N_TOKENS, _CHANNELS, 128, NC, NS, a_rows)
    xa = x_action.astype(jnp.int32)
    xi = jnp.stack([xa, x_mode.astype(jnp.int32),
                    x_readout.astype(jnp.int32), t.astype(jnp.int32)])
    wm2 = jnp.concatenate([W_mode, W_mode], axis=1)
    wr2 = jnp.concatenate([W_readout, W_readout], axis=1)
    wt2 = jnp.concatenate([W_time, W_time], axis=1)
    a_raw = k1(xa, W_action.T)
    out128 = k2(xi, a_raw, wm2, wr2, wt2)
    return out128.reshape(3, _N_TOKENS, _CHANNELS)


# speedup vs baseline: 1.4529x; 1.0022x over previous
import functools

import jax
import jax.numpy as jnp
from jax import lax
from jax.experimental import pallas as pl
from jax.experimental.pallas import tpu as pltpu
from jax.experimental.pallas import tpu_sc as plsc

_CHANNELS = 64
_N_TOKENS = 16384
_LANES = 16
_CAP = 768
_SCAP = 672
_WCAP = 288
_DUMP = _N_TOKENS


def _build_action_kernel(B, D, NC, NS):
    NW = NC * NS
    n_vec = B // _LANES
    mesh = plsc.VectorSubcoreMesh(core_axis_name="c", subcore_axis_name="s")
    out_rows = B + 8 * NW

    @functools.partial(
        pl.kernel,
        mesh=mesh,
        out_type=jax.ShapeDtypeStruct((out_rows, 2 * D), jnp.float32),
        compiler_params=pltpu.CompilerParams(needs_layout_passes=False),
        scratch_types=[
            pltpu.VMEM((B // 4,), jnp.int32),
            pltpu.VMEM((_CAP,), jnp.int32),
            pltpu.VMEM((_CAP,), jnp.int32),
            pltpu.VMEM((_CAP,), jnp.int32),
            pltpu.VMEM((_CAP,), jnp.int32),
            pltpu.VMEM((_WCAP,), jnp.int32),
            pltpu.VMEM((_WCAP,), jnp.int32),
            pltpu.VMEM((4, D, 128), jnp.float32),
            pltpu.VMEM((_SCAP, 2 * D), jnp.float32),
            pltpu.VMEM((7, 96), jnp.int32),
            pltpu.SMEM((256,), jnp.int32),
            pltpu.SemaphoreType.DMA,
            pltpu.SemaphoreType.DMA,
            pltpu.SemaphoreType.DMA,
            pltpu.SemaphoreType.DMA,
            pltpu.SemaphoreType.DMA,
        ],
    )
    def k1(xa, waT, out, av_all, xs, ps, xs2, ps2, wins, starts, W, St, ps3,
           cnt, sem, ws0, ws1, ws2, ws3):
        wid = lax.axis_index("s") * NC + lax.axis_index("c")
        wsems = [ws0, ws1, ws2, ws3]
        iota16 = jnp.arange(_LANES, dtype=jnp.int32)
        dump0 = jnp.int32(_DUMP) + wid * 8
        wbase = (wid * 32768 + 133) // 134
        lane0 = iota16 == 0

        def fill(u, _):
            ps2[pl.ds(u * _LANES, _LANES)] = jnp.broadcast_to(
                dump0 + (iota16 & 7), (_LANES,))
            return 0

        lax.fori_loop(0, _CAP // _LANES, fill, 0)

        def zero(d, _):
            cnt[d] = 0
            return 0

        lax.fori_loop(0, 256, zero, 0)
        xs2[pl.ds(0, _LANES)] = jnp.broadcast_to(jnp.int32(-1), (_LANES,))

        def p1c(c, off_c):
            cb = pl.multiple_of(c * (B // 4), B // 4)
            pltpu.sync_copy(xa.at[pl.ds(cb, B // 4)], av_all)

            def p1(u, off):
                x = av_all[pl.ds(u * _LANES, _LANES)]
                h = ((x >> 7) * 134) >> 15
                m = h == wid
                plsc.store_compressed(xs.at[pl.ds(off, _LANES)], x, mask=m)
                plsc.store_compressed(
                    ps.at[pl.ds(off, _LANES)],
                    jnp.broadcast_to(c * (B // 4) + u * _LANES, (_LANES,))
                    + iota16, mask=m)
                return off + plsc.all_reduce_population_count(m)[0]

            return lax.fori_loop(0, n_vec // 4, p1, off_c)

        n = lax.fori_loop(0, 4, p1c, jnp.int32(0))

        def hist(i, _):
            wl = (xs[pl.ds(i, _LANES)][0] >> 7) - wbase
            cnt[wl] = cnt[wl] + 1
            return 0

        lax.fori_loop(0, n, hist, 0)

        def pfx(d, run):
            c = cnt[d]
            cnt[d] = run
            return run + c

        lax.fori_loop(0, 256, pfx, jnp.int32(0))

        def sca(i, _):
            x = xs[pl.ds(i, _LANES)][0]
            p = ps[pl.ds(i, _LANES)][0]
            wl = (x >> 7) - wbase
            o = cnt[wl] + _LANES
            cnt[wl] = o + 1 - _LANES
            plsc.store_compressed(
                xs2.at[pl.ds(o, _LANES)],
                jnp.broadcast_to(x, (_LANES,)), mask=lane0)
            plsc.store_compressed(
                ps2.at[pl.ds(o, _LANES)],
                jnp.broadcast_to(p, (_LANES,)), mask=lane0)
            return 0

        lax.fori_loop(0, n, sca, 0)

        def p2(u, off):
            idx = u * _LANES + iota16
            w = xs2[pl.ds(u * _LANES, _LANES)] >> 7
            sel = jnp.maximum(u * _LANES - 1, 0)
            wm1 = xs2[pl.ds(sel, _LANES)] >> 7
            valid = jnp.logical_and(idx >= _LANES, idx < n + _LANES)
            bm = jnp.logical_and(w != wm1, valid)
            plsc.store_compressed(wins.at[pl.ds(off, _LANES)], w, mask=bm)
            plsc.store_compressed(
                starts.at[pl.ds(off, _LANES)],
                jnp.broadcast_to(jnp.int32(u * _LANES), (_LANES,)) + iota16,
                mask=bm)
            return off + plsc.all_reduce_population_count(bm)[0]

        nw = lax.fori_loop(0, _CAP // _LANES, p2, jnp.int32(0))
        starts[pl.ds(nw, _LANES)] = jnp.broadcast_to(n + _LANES, (_LANES,))

        def fire(v, slot):
            wv = wins[pl.ds(v, _LANES)][0]
            s = pl.multiple_of(wv * 128, 128)
            pltpu.async_copy(waT.at[:, pl.ds(s, 128)], W.at[slot],
                             wsems[slot])

        def drain(slot):
            pltpu.make_async_copy(
                waT.at[:, pl.ds(0, 128)], W.at[slot], wsems[slot]).wait()

        def proc(v, slot):
            sv = starts[pl.ds(v, _LANES)]
            s0, s1 = sv[0], sv[1]

            def tok(i, _):
                x = xs2[pl.ds(i, _LANES)][0]
                colv = jnp.broadcast_to(x & 127, (_LANES,))
                for j in range(D // _LANES):
                    av = plsc.load_gather(
                        W.at[slot], [iota16 + j * _LANES, colv])
                    St[i - _LANES, pl.ds(j * _LANES, _LANES)] = av
                return 0

            lax.fori_loop(s0, s1, tok, 0)

        for b in range(4):
            @pl.when(nw > b)
            def _pro(b=b):
                fire(b, b)

        def wbody(vv, _):
            v0 = vv * 4
            drain(0)
            proc(v0, 0)

            @pl.when(v0 + 4 < nw)
            def _f0():
                fire(v0 + 4, 0)

            for b in range(1, 4):
                @pl.when(v0 + b < nw)
                def _sb(b=b):
                    drain(b)
                    proc(v0 + b, b)

                    @pl.when(v0 + b + 4 < nw)
                    def _fb(b=b):
                        fire(v0 + b + 4, b)

            return 0

        lax.fori_loop(0, (nw + 3) // 4, wbody, 0)

        for j in range(7):
            for m in range(6):
                ps3[j, pl.ds(m * _LANES, _LANES)] = (
                    ps2[pl.ds(_LANES + 96 * j + m * _LANES, _LANES)])

        cps = [
            pltpu.async_copy(St.at[pl.ds(96 * j, 96)], out.at[ps3.at[j]],
                             sem)
            for j in range(7)
        ]
        for cp in cps:
            cp.wait()

    return k1


def _build_assemble_kernel(B, D, C, NC, NS, a_rows):
    NW = NC * NS
    per_w = B // NW
    n_chunks = per_w // C
    mesh = plsc.VectorSubcoreMesh(core_axis_name="c", subcore_axis_name="s")

    @functools.partial(
        pl.kernel,
        mesh=mesh,
        out_type=jax.ShapeDtypeStruct((3, B // 2, 2 * D), jnp.float32),
        compiler_params=pltpu.CompilerParams(needs_layout_passes=False),
        scratch_types=[
            pltpu.VMEM((2, 4, C), jnp.int32),
            pltpu.VMEM((2, C, 2 * D), jnp.float32),
            pltpu.VMEM((2, C, 2 * D), jnp.float32),
            pltpu.VMEM((2, C, 2 * D), jnp.float32),
            pltpu.VMEM((4, 2 * D), jnp.float32),
            pltpu.VMEM((C // 2, 2 * D), jnp.float32),
            pltpu.VMEM((C // 2, 2 * D), jnp.float32),
            pltpu.VMEM((C // 2, 2 * D), jnp.float32),
            pltpu.SemaphoreType.DMA,
            pltpu.SemaphoreType.DMA,
        ],
    )
    def k2(xi, a_raw, wm2, wr2, wt2, out,
           ix, A, M, T, Rt, As, Ms, Rs, sem0, sem1):
        wid = lax.axis_index("s") * NC + lax.axis_index("c")
        base0 = wid * per_w
        sems = [sem0, sem1]
        pltpu.sync_copy(wr2, Rt)

        def stage(ci, p):
            base = pl.multiple_of(base0 + ci * C, C)
            pltpu.sync_copy(xi.at[:, pl.ds(base, C)], ix.at[p])
            pltpu.async_copy(a_raw.at[pl.ds(base, C)], A.at[p], sems[p])
            pltpu.async_copy(wm2.at[ix.at[p, 1]], M.at[p], sems[p])
            pltpu.async_copy(wt2.at[ix.at[p, 3]], T.at[p], sems[p])

        def wait3(p):
            pltpu.make_async_copy(
                a_raw.at[pl.ds(0, C)], A.at[p], sems[p]).wait()
            pltpu.make_async_copy(
                a_raw.at[pl.ds(0, C)], M.at[p], sems[p]).wait()
            pltpu.make_async_copy(
                a_raw.at[pl.ds(0, C)], T.at[p], sems[p]).wait()

        stage(0, 0)
        for ci in range(n_chunks):
            p = ci % 2
            if ci + 1 < n_chunks:
                stage(ci + 1, 1 - p)
            wait3(p)

            def row(g, _2):
                xrs = ix[p, 2, pl.ds(g * _LANES, _LANES)]
                for l in range(_LANES):
                    i = g * _LANES + l
                    srow = i // 2
                    soff = D * (l & 1)
                    rrow = xrs[l]
                    for j in range(D // _LANES):
                        sl = pl.ds(j * _LANES, _LANES)
                        tv = T[p, i, sl]
                        av = A[p, i, sl]
                        mv = M[p, i, sl]
                        rv = Rt[rrow, sl]
                        dsl = pl.ds(soff + j * _LANES, _LANES)
                        As[srow, dsl] = av + tv
                        Ms[srow, dsl] = mv + tv
                        Rs[srow, dsl] = rv + tv
                return 0

            lax.fori_loop(0, C // _LANES, row, 0)
            base = pl.multiple_of(base0 + ci * C, C)
            hbase = pl.multiple_of(base // 2, C // 2)
            pltpu.sync_copy(As, out.at[0, pl.ds(hbase, C // 2)])
            pltpu.sync_copy(Ms, out.at[1, pl.ds(hbase, C // 2)])
            pltpu.sync_copy(Rs, out.at[2, pl.ds(hbase, C // 2)])

    return k2


def kernel(x_action, x_mode, x_readout, t, W_action, W_mode, W_readout, W_time):
    info = plsc.get_sparse_core_info()
    NC, NS = info.num_cores, info.num_subcores
    a_rows = _N_TOKENS + 8 * NC * NS
    k1 = _build_action_kernel(_N_TOKENS, _CHANNELS, NC, NS)
    k2 = _build_assemble_kernel(_N_TOKENS, _CHANNELS, 128, NC, NS, a_rows)
    xa = x_action.astype(jnp.int32)
    xi = jnp.stack([xa, x_mode.astype(jnp.int32),
                    x_readout.astype(jnp.int32), t.astype(jnp.int32)])
    wm2 = jnp.concatenate([W_mode, W_mode], axis=1)
    wr2 = jnp.concatenate([W_readout, W_readout], axis=1)
    wt2 = jnp.concatenate([W_time, W_time], axis=1)
    a_raw = k1(xa, W_action.T)
    out128 = k2(xi, a_raw, wm2, wr2, wt2)
    return out128.reshape(3, _N_TOKENS, _CHANNELS)
